# Initial kernel scaffold; baseline (speedup 1.0000x reference)
import jax
import jax.numpy as jnp
from jax.experimental import pallas as pl


def kernel(x, edge_attr, edge_weight, edge_dst, edge_src):
    raise NotImplementedError("write your pallas kernel here")


# speedup vs baseline: 3.0086x; 3.0086x over previous
import functools

import jax
import jax.numpy as jnp
from jax import lax
from jax.experimental import pallas as pl
from jax.experimental.pallas import tpu as pltpu
from jax.experimental.pallas import tpu_sc as plsc

N = 10000
E = 160000
D = 128
V = 4
ALPHA = (1.0 / V) ** 0.5

NC = 2
NS = 16
NW = NC * NS
S = 128
NCHUNKS = E // S
BASE_CH = NCHUNKS // NW
EXTRA = NCHUNKS - BASE_CH * NW

BE = 640


def _contract_body(attr_ref, w_ref, t_ref):
    a = attr_ref[...]
    w = w_ref[...]
    j = lax.broadcasted_iota(jnp.int32, (V, D * V), 1)
    v = lax.broadcasted_iota(jnp.int32, (V, D * V), 0)
    expand = (j % V == v).astype(jnp.float32)
    at = lax.dot(a, expand, preferred_element_type=jnp.float32)
    p = w * at
    jj = lax.broadcasted_iota(jnp.int32, (D * V, D), 0)
    uu = lax.broadcasted_iota(jnp.int32, (D * V, D), 1)
    red = jnp.where(jj // V == uu, jnp.float32(ALPHA), jnp.float32(0.0))
    t_ref[...] = lax.dot(p, red, preferred_element_type=jnp.float32)


def _contract(edge_attr, edge_weight):
    return pl.pallas_call(
        _contract_body,
        grid=(E // BE,),
        in_specs=[
            pl.BlockSpec((BE, V), lambda i: (i, 0)),
            pl.BlockSpec((BE, D * V), lambda i: (i, 0)),
        ],
        out_specs=pl.BlockSpec((BE, D), lambda i: (i, 0)),
        out_shape=jax.ShapeDtypeStruct((E, D), jnp.float32),
    )(edge_attr, edge_weight)


ROWS_A = 624
ROWS_B = N - ROWS_A * (NS - 1)


def _sc_scatter(x, tflat, src1, dst1, zeros):
    mesh = plsc.VectorSubcoreMesh(
        core_axis_name="c", subcore_axis_name="s", num_cores=NC, num_subcores=NS
    )

    @functools.partial(
        pl.kernel,
        out_type=jax.ShapeDtypeStruct((NC, N, D), jnp.float32),
        mesh=mesh,
        scratch_types=[
            pltpu.VMEM((S,), jnp.int32),
            pltpu.VMEM((S,), jnp.int32),
            pltpu.VMEM((S * D,), jnp.float32),
            pltpu.VMEM((S, D), jnp.float32),
            pltpu.VMEM_SHARED((N, D), jnp.float32),
            pltpu.SemaphoreType.DMA,
        ],
    )
    def body(x_hbm, t_hbm, src_hbm, dst_hbm, z_hbm, out_hbm,
             src_v, dst_v, t_v, xs_v, acc, sem):
        c = lax.axis_index("c")
        s = lax.axis_index("s")
        wid = s * NC + c
        r0 = pl.multiple_of(s * ROWS_A, 8)

        @pl.when(s < NS - 1)
        def _():
            pltpu.sync_copy(z_hbm.at[pl.ds(0, ROWS_A)], acc.at[pl.ds(r0, ROWS_A)])

        @pl.when(s == NS - 1)
        def _():
            pltpu.sync_copy(z_hbm, acc.at[pl.ds(r0, ROWS_B)])

        plsc.subcore_barrier()

        start = wid * BASE_CH + jnp.minimum(wid, EXTRA)
        nk = BASE_CH + jnp.where(wid < EXTRA, 1, 0)

        def chunk(k, _):
            ch = start + k
            pltpu.sync_copy(src_hbm.at[pl.ds(pl.multiple_of(ch * S, S), S)], src_v)
            pltpu.sync_copy(dst_hbm.at[pl.ds(pl.multiple_of(ch * S, S), S)], dst_v)
            pltpu.sync_copy(
                t_hbm.at[pl.ds(pl.multiple_of(ch * (S * D), S * D), S * D)], t_v)
            pltpu.async_copy(x_hbm.at[src_v], xs_v, sem).wait()

            def mul(e, _):
                base = e * D
                for j in range(D // 16):
                    sl = pl.ds(j * 16, 16)
                    xs_v[e, sl] = xs_v[e, sl] * t_v[pl.ds(base + j * 16, 16)]
                return 0
            lax.fori_loop(0, S, mul, 0)
            pltpu.sync_copy(xs_v, acc.at[dst_v], add=True)
            return 0
        lax.fori_loop(0, nk, chunk, 0)
        plsc.subcore_barrier()

        @pl.when(s < NS - 1)
        def _():
            pltpu.sync_copy(acc.at[pl.ds(r0, ROWS_A)],
                            out_hbm.at[c, pl.ds(r0, ROWS_A)])

        @pl.when(s == NS - 1)
        def _():
            pltpu.sync_copy(acc.at[pl.ds(r0, ROWS_B)],
                            out_hbm.at[c, pl.ds(r0, ROWS_B)])

    return body(x, tflat, src1, dst1, zeros)


def _combine_body(p_ref, o_ref):
    o_ref[...] = p_ref[0] + p_ref[1]


def _combine(partials):
    rb = 1000
    return pl.pallas_call(
        _combine_body,
        grid=(N // rb,),
        in_specs=[pl.BlockSpec((NC, rb, D), lambda i: (0, i, 0))],
        out_specs=pl.BlockSpec((rb, D), lambda i: (i, 0)),
        out_shape=jax.ShapeDtypeStruct((N, D), jnp.float32),
    )(partials)


def kernel(x, edge_attr, edge_weight, edge_dst, edge_src):
    t = _contract(edge_attr, edge_weight)
    tflat = t.reshape(E * D)
    zeros = jnp.zeros((ROWS_B, D), jnp.float32)
    partials = _sc_scatter(x, tflat, edge_src.astype(jnp.int32),
                           edge_dst.astype(jnp.int32), zeros)
    return _combine(partials)


# speedup vs baseline: 3.5733x; 1.1877x over previous
import functools

import jax
import jax.numpy as jnp
from jax import lax
from jax.experimental import pallas as pl
from jax.experimental.pallas import tpu as pltpu
from jax.experimental.pallas import tpu_sc as plsc

N = 10000
E = 160000
D = 128
V = 4
ALPHA = (1.0 / V) ** 0.5

NC = 2
NS = 16
NW = NC * NS
S = 64
NCHUNKS = E // S
BASE_CH = NCHUNKS // NW
EXTRA = NCHUNKS - BASE_CH * NW

BE = 640


def _contract_body(attr_ref, w_ref, t_ref):
    a = attr_ref[...]
    w = w_ref[...]
    j = lax.broadcasted_iota(jnp.int32, (V, D * V), 1)
    v = lax.broadcasted_iota(jnp.int32, (V, D * V), 0)
    expand = (j % V == v).astype(jnp.float32)
    at = lax.dot(a, expand, preferred_element_type=jnp.float32)
    p = w * at
    jj = lax.broadcasted_iota(jnp.int32, (D * V, D), 0)
    uu = lax.broadcasted_iota(jnp.int32, (D * V, D), 1)
    red = jnp.where(jj // V == uu, jnp.float32(ALPHA), jnp.float32(0.0))
    t_ref[...] = lax.dot(p, red, preferred_element_type=jnp.float32)


def _contract(edge_attr, edge_weight):
    return pl.pallas_call(
        _contract_body,
        grid=(E // BE,),
        in_specs=[
            pl.BlockSpec((BE, V), lambda i: (i, 0)),
            pl.BlockSpec((BE, D * V), lambda i: (i, 0)),
        ],
        out_specs=pl.BlockSpec((BE, D), lambda i: (i, 0)),
        out_shape=jax.ShapeDtypeStruct((E, D), jnp.float32),
    )(edge_attr, edge_weight)


ROWS_A = 624
ROWS_B = N - ROWS_A * (NS - 1)


def _sc_scatter(x, tflat, src1, dst1, zeros):
    mesh = plsc.VectorSubcoreMesh(
        core_axis_name="c", subcore_axis_name="s", num_cores=NC, num_subcores=NS
    )

    NB = 3

    @functools.partial(
        pl.kernel,
        out_type=jax.ShapeDtypeStruct((NC, N, D), jnp.float32),
        mesh=mesh,
        scratch_types=(
            [pltpu.VMEM((S,), jnp.int32) for _ in range(NB)]
            + [pltpu.VMEM((S,), jnp.int32) for _ in range(NB)]
            + [pltpu.VMEM((S * D,), jnp.float32) for _ in range(NB)]
            + [pltpu.VMEM((S, D), jnp.float32) for _ in range(NB)]
            + [pltpu.VMEM_SHARED((N, D), jnp.float32)]
            + [pltpu.SemaphoreType.DMA for _ in range(3 * NB)]
        ),
    )
    def body(x_hbm, t_hbm, src_hbm, dst_hbm, z_hbm, out_hbm, *scr):
        src_v = scr[0:NB]
        dst_v = scr[NB:2 * NB]
        t_v = scr[2 * NB:3 * NB]
        xs_v = scr[3 * NB:4 * NB]
        acc = scr[4 * NB]
        sem_in = scr[4 * NB + 1:4 * NB + 1 + NB]
        sem_g = scr[4 * NB + 1 + NB:4 * NB + 1 + 2 * NB]
        sem_sc = scr[4 * NB + 1 + 2 * NB:4 * NB + 1 + 3 * NB]

        c = lax.axis_index("c")
        s = lax.axis_index("s")
        wid = s * NC + c
        r0 = pl.multiple_of(s * ROWS_A, 8)

        @pl.when(s < NS - 1)
        def _():
            pltpu.sync_copy(z_hbm.at[pl.ds(0, ROWS_A)], acc.at[pl.ds(r0, ROWS_A)])

        @pl.when(s == NS - 1)
        def _():
            pltpu.sync_copy(z_hbm, acc.at[pl.ds(r0, ROWS_B)])

        plsc.subcore_barrier()

        start = wid * BASE_CH + jnp.minimum(wid, EXTRA)
        nk = BASE_CH + jnp.where(wid < EXTRA, 1, 0)

        def in_copies(k, b):
            ch = start + k
            eo = pl.ds(pl.multiple_of(ch * S, S), S)
            to = pl.ds(pl.multiple_of(ch * (S * D), S * D), S * D)
            return (
                pltpu.make_async_copy(src_hbm.at[eo], src_v[b], sem_in[b]),
                pltpu.make_async_copy(dst_hbm.at[eo], dst_v[b], sem_in[b]),
                pltpu.make_async_copy(t_hbm.at[to], t_v[b], sem_in[b]),
            )

        def step(k, b):
            b1 = (b + 1) % NB
            b2 = (b + 2) % NB

            @pl.when(k + 1 < nk)
            def _():
                for cp in in_copies(k + 1, b1):
                    cp.wait()
                pltpu.async_copy(x_hbm.at[src_v[b1]], xs_v[b1], sem_g[b1])

            @pl.when(k < nk)
            def _():
                pltpu.make_async_copy(x_hbm.at[src_v[b]], xs_v[b], sem_g[b]).wait()

                def mul(e, _):
                    for j in range(D // 16):
                        sl = pl.ds(j * 16, 16)
                        xs_v[b][e, sl] = (
                            xs_v[b][e, sl] * t_v[b][pl.ds(e * D + j * 16, 16)])
                    return 0
                lax.fori_loop(0, S, mul, 0)

                pltpu.async_copy(xs_v[b], acc.at[dst_v[b]], sem_sc[b], add=True)

            @pl.when(jnp.logical_and(k >= 1, k < nk))
            def _():
                pltpu.make_async_copy(
                    xs_v[(b + NB - 1) % NB],
                    acc.at[dst_v[(b + NB - 1) % NB]],
                    sem_sc[(b + NB - 1) % NB]).wait()

            @pl.when(k + 2 < nk)
            def _():
                for cp in in_copies(k + 2, b2):
                    cp.start()

        for cp in in_copies(0, 0):
            cp.start()
        for cp in in_copies(1, 1):
            cp.start()
        for cp in in_copies(0, 0):
            cp.wait()
        pltpu.async_copy(x_hbm.at[src_v[0]], xs_v[0], sem_g[0])

        n_outer = -(-(BASE_CH + 1) // NB)

        def outer(k3, _):
            for i in range(NB):
                step(k3 * NB + i, i)
            return 0
        lax.fori_loop(0, n_outer, outer, 0)

        for nkv in (BASE_CH, BASE_CH + 1):
            @pl.when(nk == nkv)
            def _():
                bl = (nkv - 1) % NB
                pltpu.make_async_copy(
                    xs_v[bl], acc.at[dst_v[bl]], sem_sc[bl]).wait()

        plsc.subcore_barrier()

        @pl.when(s < NS - 1)
        def _():
            pltpu.sync_copy(acc.at[pl.ds(r0, ROWS_A)],
                            out_hbm.at[c, pl.ds(r0, ROWS_A)])

        @pl.when(s == NS - 1)
        def _():
            pltpu.sync_copy(acc.at[pl.ds(r0, ROWS_B)],
                            out_hbm.at[c, pl.ds(r0, ROWS_B)])

    return body(x, tflat, src1, dst1, zeros)


def _combine_body(p_ref, o_ref):
    o_ref[...] = p_ref[0] + p_ref[1]


def _combine(partials):
    rb = 1000
    return pl.pallas_call(
        _combine_body,
        grid=(N // rb,),
        in_specs=[pl.BlockSpec((NC, rb, D), lambda i: (0, i, 0))],
        out_specs=pl.BlockSpec((rb, D), lambda i: (i, 0)),
        out_shape=jax.ShapeDtypeStruct((N, D), jnp.float32),
    )(partials)


def kernel(x, edge_attr, edge_weight, edge_dst, edge_src):
    t = _contract(edge_attr, edge_weight)
    tflat = t.reshape(E * D)
    zeros = jnp.zeros((ROWS_B, D), jnp.float32)
    partials = _sc_scatter(x, tflat, edge_src.astype(jnp.int32),
                           edge_dst.astype(jnp.int32), zeros)
    return _combine(partials)
